# baseline (speedup 1.0000x reference)
import functools

import jax
import jax.numpy as jnp
from jax import lax
from jax.experimental import pallas as pl
from jax.experimental.pallas import tpu as pltpu
from jax.experimental.pallas import tpu_sc as plsc

_B = 16384
_D = 32
_FEAT = 102
_OUT = 64
_BBLK = 2048
_CHUNK = 128
_NROWS4 = 250000


def _make_sc_gather():
    info = plsc.get_sparse_core_info()
    nc, ns = info.num_cores, info.num_subcores
    nw = nc * ns
    rpw = _B // nw
    n_chunks = rpw // _CHUNK
    mesh = plsc.VectorSubcoreMesh(core_axis_name="c", subcore_axis_name="s")

    @functools.partial(
        pl.kernel,
        mesh=mesh,
        out_type=jax.ShapeDtypeStruct((_D, _B), jnp.float32),
        scratch_types=[
            pltpu.VMEM((n_chunks, _CHUNK), jnp.int32),
            pltpu.VMEM((n_chunks, _CHUNK), jnp.int32),
            pltpu.VMEM((rpw, _CHUNK), jnp.float32),
            pltpu.VMEM((_D, rpw), jnp.float32),
            pltpu.SemaphoreType.DMA,
        ],
        compiler_params=pltpu.CompilerParams(needs_layout_passes=False),
    )
    def sc_gather(nt4_hbm, q_hbm, lane_hbm, outT_hbm, q_v, lane_v, grp_v,
                  rowsT_v, sem):
        wid = lax.axis_index("s") * nc + lax.axis_index("c")
        base = wid * rpw
        pltpu.sync_copy(q_hbm.at[pl.ds(wid * n_chunks, n_chunks)], q_v)
        pltpu.sync_copy(lane_hbm.at[pl.ds(wid * n_chunks, n_chunks)], lane_v)
        for j in range(n_chunks):
            pltpu.async_copy(
                nt4_hbm.at[q_v.at[j]],
                grp_v.at[pl.ds(j * _CHUNK, _CHUNK)],
                sem,
            )
        for j in range(n_chunks):
            pltpu.make_async_copy(
                nt4_hbm.at[pl.ds(0, _CHUNK)],
                grp_v.at[pl.ds(j * _CHUNK, _CHUNK)],
                sem,
            ).wait()

        def extract(g, carry):
            j = g // (_CHUNK // 16)
            k = g % (_CHUNK // 16)
            i0 = g * 16
            lane0 = lane_v[j, pl.ds(k * 16, 16)]
            rows16 = i0 + lax.iota(jnp.int32, 16)
            for c in range(_D):
                vals = plsc.load_gather(grp_v, [rows16, lane0 + c])
                rowsT_v[c, pl.ds(i0, 16)] = vals
            return carry

        lax.fori_loop(0, rpw // 16, extract, 0)
        pltpu.sync_copy(rowsT_v, outT_hbm.at[:, pl.ds(base, rpw)])

    return sc_gather, nw * n_chunks


_SC_GATHER, _IDX_ROWS = _make_sc_gather()


def _tc_a_body(x_ref, c_ref, b_ref, o_ref):
    xb = x_ref[...]
    ri = lax.broadcasted_iota(jnp.int32, xb.shape, 0)
    valid = (ri >= 10) & (ri <= 92)
    xm = jnp.where(valid, xb, jnp.int32(-2147483648))
    m = jnp.max(xm, axis=0, keepdims=True)
    iidx = jnp.min(jnp.where(xm == m, ri, _FEAT), axis=0, keepdims=True)
    onehot = (ri == iidx).astype(jnp.float32)
    o_ref[...] = (
        jnp.dot(c_ref[...], onehot, preferred_element_type=jnp.float32)
        + b_ref[...]
    )


def _tc_b_body(p_ref, ne_ref, w_ref, o_ref):
    o_ref[...] = (
        jnp.dot(w_ref[...], ne_ref[...], preferred_element_type=jnp.float32)
        + p_ref[...]
    )


@jax.jit
def kernel(X, numeric_table, i_table, W_fc, b_fc):
    Xt = X.T
    itT_pad = jnp.pad(i_table.T, ((0, 0), (10, _FEAT - 93)))
    Wi = W_fc[:, _D:]
    Wn = W_fc[:, :_D]
    C = Wi @ itT_pad + jnp.concatenate(
        [jnp.zeros((_D, _FEAT), jnp.float32), itT_pad], axis=0
    )
    Wn_aug = Wn + jnp.concatenate(
        [jnp.eye(_D, dtype=jnp.float32), jnp.zeros((_D, _D), jnp.float32)],
        axis=0,
    )

    nt4 = numeric_table.reshape(_NROWS4, 4 * _D)
    idx = X[:, 9]
    q = lax.shift_right_logical(idx, 2).reshape(_IDX_ROWS, _CHUNK)
    lane = (jnp.bitwise_and(idx, 3) * _D).reshape(_IDX_ROWS, _CHUNK)
    neT = _SC_GATHER(nt4, q, lane)

    grid = _B // _BBLK
    partialT = pl.pallas_call(
        _tc_a_body,
        grid=(grid,),
        in_specs=[
            pl.BlockSpec((_FEAT, _BBLK), lambda i: (0, i)),
            pl.BlockSpec((_OUT, _FEAT), lambda i: (0, 0)),
            pl.BlockSpec((_OUT, 1), lambda i: (0, 0)),
        ],
        out_specs=pl.BlockSpec((_OUT, _BBLK), lambda i: (0, i)),
        out_shape=jax.ShapeDtypeStruct((_OUT, _B), jnp.float32),
    )(Xt, C, b_fc.reshape(_OUT, 1))

    outT = pl.pallas_call(
        _tc_b_body,
        grid=(grid,),
        in_specs=[
            pl.BlockSpec((_OUT, _BBLK), lambda i: (0, i)),
            pl.BlockSpec((_D, _BBLK), lambda i: (0, i)),
            pl.BlockSpec((_OUT, _D), lambda i: (0, 0)),
        ],
        out_specs=pl.BlockSpec((_OUT, _BBLK), lambda i: (0, i)),
        out_shape=jax.ShapeDtypeStruct((_OUT, _B), jnp.float32),
    )(partialT, neT, Wn_aug)
    return outT.T


# speedup vs baseline: 1.4797x; 1.4797x over previous
import functools

import jax
import jax.numpy as jnp
from jax import lax
from jax.experimental import pallas as pl
from jax.experimental.pallas import tpu as pltpu
from jax.experimental.pallas import tpu_sc as plsc

_B = 16384
_D = 32
_FEAT = 102
_OUT = 64
_BBLK = 2048
_WAVE = 16


def _make_sc_gather():
    info = plsc.get_sparse_core_info()
    nc, ns = info.num_cores, info.num_subcores
    nw = nc * ns
    rpw = _B // nw
    nwaves = rpw // _WAVE
    mesh = plsc.VectorSubcoreMesh(core_axis_name="c", subcore_axis_name="s")

    @functools.partial(
        pl.kernel,
        mesh=mesh,
        out_type=jax.ShapeDtypeStruct((_D, _B), jnp.float32),
        scratch_types=[
            pltpu.VMEM((1, rpw), jnp.int32),
            pltpu.VMEM((1, rpw), jnp.int32),
            pltpu.VMEM((64, 8, _D), jnp.float32),
            pltpu.VMEM((_D, 128), jnp.float32),
            pltpu.SemaphoreType.DMA,
        ],
        compiler_params=pltpu.CompilerParams(needs_layout_passes=False),
    )
    def sc_gather(nt_hbm, g8_hbm, sub_hbm, outT_hbm, g8_v, sub_v,
                  grp_v, rowsT_v, sem):
        wid = lax.axis_index("s") * nc + lax.axis_index("c")
        base = wid * rpw
        pltpu.sync_copy(g8_hbm.at[pl.ds(wid, 1)], g8_v)
        pltpu.sync_copy(sub_hbm.at[pl.ds(wid, 1)], sub_v)

        round_n = 64

        def fire(t, off):
            g8vec = g8_v[0, pl.ds(off + t * _WAVE, _WAVE)]
            for j in range(_WAVE):
                i = t * _WAVE + j
                r8 = pl.multiple_of(g8vec[j], 8)
                pltpu.async_copy(
                    nt_hbm.at[pl.ds(r8, 8), :],
                    grp_v.at[i],
                    sem,
                )

        def drain():
            for _ in range(_WAVE):
                pltpu.make_async_copy(
                    nt_hbm.at[pl.ds(0, 8), :],
                    grp_v.at[0],
                    sem,
                ).wait()

        def extract(g, off, col0):
            i0 = g * 16
            subs = sub_v[0, pl.ds(off + i0, 16)]
            rows16 = i0 + lax.iota(jnp.int32, 16)
            for c in range(_D):
                vals = plsc.load_gather(
                    grp_v, [rows16, subs, jnp.full((16,), c, jnp.int32)]
                )
                rowsT_v[c, pl.ds(col0 + i0, 16)] = vals

        for round_id in range(rpw // round_n):
            off = round_id * round_n
            col0 = (round_id % 2) * round_n

            def body(t, carry):
                fire(t, off)
                drain()
                return carry

            fire(0, off)
            lax.fori_loop(1, round_n // _WAVE, body, 0, unroll=False)
            drain()

            def extract_body(g, carry):
                extract(g, off, col0)
                return carry

            lax.fori_loop(0, round_n // 16, extract_body, 0, unroll=False)
            if round_id % 2 == 1:
                pltpu.sync_copy(
                    rowsT_v,
                    outT_hbm.at[:, pl.ds(base + (round_id - 1) * round_n, 128)],
                )

    return sc_gather, nw


_SC_GATHER, _NW = _make_sc_gather()


def _tc_a_body(x_ref, c_ref, b_ref, o_ref):
    xb = x_ref[...]
    ri = lax.broadcasted_iota(jnp.int32, xb.shape, 0)
    valid = (ri >= 10) & (ri <= 92)
    xm = jnp.where(valid, xb, jnp.int32(-2147483648))
    m = jnp.max(xm, axis=0, keepdims=True)
    iidx = jnp.min(jnp.where(xm == m, ri, _FEAT), axis=0, keepdims=True)
    onehot = (ri == iidx).astype(jnp.float32)
    o_ref[...] = (
        jnp.dot(c_ref[...], onehot, preferred_element_type=jnp.float32)
        + b_ref[...]
    )


def _tc_b_body(p_ref, ne_ref, w_ref, o_ref):
    o_ref[...] = (
        jnp.dot(w_ref[...], ne_ref[...], preferred_element_type=jnp.float32)
        + p_ref[...]
    )


@jax.jit
def kernel(X, numeric_table, i_table, W_fc, b_fc):
    Xt = X.T
    itT_pad = jnp.pad(i_table.T, ((0, 0), (10, _FEAT - 93)))
    Wi = W_fc[:, _D:]
    Wn = W_fc[:, :_D]
    C = Wi @ itT_pad + jnp.concatenate(
        [jnp.zeros((_D, _FEAT), jnp.float32), itT_pad], axis=0
    )
    Wn_aug = Wn + jnp.concatenate(
        [jnp.eye(_D, dtype=jnp.float32), jnp.zeros((_D, _D), jnp.float32)],
        axis=0,
    )

    idx = X[:, 9]
    g8 = (lax.shift_right_logical(idx, 3) * 8).reshape(_NW, _B // _NW)
    sub = jnp.bitwise_and(idx, 7).reshape(_NW, _B // _NW)
    neT = _SC_GATHER(numeric_table, g8, sub)

    grid = _B // _BBLK
    partialT = pl.pallas_call(
        _tc_a_body,
        grid=(grid,),
        in_specs=[
            pl.BlockSpec((_FEAT, _BBLK), lambda i: (0, i)),
            pl.BlockSpec((_OUT, _FEAT), lambda i: (0, 0)),
            pl.BlockSpec((_OUT, 1), lambda i: (0, 0)),
        ],
        out_specs=pl.BlockSpec((_OUT, _BBLK), lambda i: (0, i)),
        out_shape=jax.ShapeDtypeStruct((_OUT, _B), jnp.float32),
    )(Xt, C, b_fc.reshape(_OUT, 1))

    outT = pl.pallas_call(
        _tc_b_body,
        grid=(grid,),
        in_specs=[
            pl.BlockSpec((_OUT, _BBLK), lambda i: (0, i)),
            pl.BlockSpec((_D, _BBLK), lambda i: (0, i)),
            pl.BlockSpec((_OUT, _D), lambda i: (0, 0)),
        ],
        out_specs=pl.BlockSpec((_OUT, _BBLK), lambda i: (0, i)),
        out_shape=jax.ShapeDtypeStruct((_OUT, _B), jnp.float32),
    )(partialT, neT, Wn_aug)
    return outT.T
